# baseline (speedup 1.0000x reference)
import jax
import jax.numpy as jnp
from jax import lax
from jax.experimental import pallas as pl
from jax.experimental.pallas import tpu as pltpu
from jax.experimental.pallas import tpu_sc as plsc

NUM_EMB = 1000000
DIM = 128
B = 4096
L = 50
N = B * L

NC, NS = 2, 16
NW = NC * NS
K = 5
NK = N // K
ROWS_PER_W = NK // NW
CHUNK = 128
NCHUNK = ROWS_PER_W // CHUNK
NBUF = 5
LAG = NBUF - 1


def _sc_gather(ids_hbm, table_hbm, out_hbm, idx_v, rows_v, bf_v, g_sem, s_sem):
    wid = lax.axis_index("s") * NC + lax.axis_index("c")
    base2 = wid * (ROWS_PER_W // 2)
    pltpu.sync_copy(ids_hbm.at[wid], idx_v)
    g_h = [None] * NCHUNK
    s_h = [None] * NCHUNK

    def _convert(k):
        def body(rr, carry):
            for g in range(DIM // 16):
                a = rows_v[k, 2 * rr, pl.ds(16 * g, 16)]
                bb = rows_v[k, 2 * rr + 1, pl.ds(16 * g, 16)]
                y = plsc.pack(a, bb, format=plsc.PackFormat.INTERLEAVED)
                bf_v[k, rr, pl.ds(16 * g, 16)] = plsc.bitcast(y, jnp.float32)
            return carry

        lax.fori_loop(0, CHUNK // 2, body, None)

    for c in range(NCHUNK + LAG):
        if c < NCHUNK:
            g_h[c] = pltpu.async_copy(
                table_hbm.at[idx_v.at[c]], rows_v.at[c % NBUF], g_sem
            )
        d = c - LAG
        if 0 <= d < NCHUNK:
            if d >= NBUF:
                s_h[d - NBUF].wait()
            g_h[d].wait()
            _convert(d % NBUF)
            s_h[d] = pltpu.async_copy(
                bf_v.at[d % NBUF],
                out_hbm.at[pl.ds(base2 + d * (CHUNK // 2), CHUNK // 2)],
                s_sem,
            )
    for d in range(max(NCHUNK - NBUF, 0), NCHUNK):
        s_h[d].wait()


def _gather_call(ids, table):
    return pl.kernel(
        _sc_gather,
        mesh=plsc.VectorSubcoreMesh(
            core_axis_name="c", subcore_axis_name="s", num_cores=NC
        ),
        out_type=jax.ShapeDtypeStruct((NK // 2, DIM), jnp.float32),
        scratch_types=[
            pltpu.VMEM((NCHUNK, CHUNK), jnp.int32),
            pltpu.VMEM((NBUF, CHUNK, DIM), jnp.float32),
            pltpu.VMEM((NBUF, CHUNK // 2, DIM), jnp.float32),
            pltpu.SemaphoreType.DMA,
            pltpu.SemaphoreType.DMA,
        ],
        compiler_params=pltpu.CompilerParams(needs_layout_passes=False),
    )(ids, table)


MM_BLK2 = 2048
MM_STEPS = (NK // 2) // MM_BLK2


def _mm_math(emb_ref, wt_ref, b_ref):
    u = pltpu.bitcast(emb_ref[...], jnp.bfloat16)
    return (
        jnp.dot(u, wt_ref[...], preferred_element_type=jnp.float32) + b_ref[...]
    )


def _mm_body(prev_ref, emb_ref, wt_ref, b_ref, out_ref):
    del prev_ref
    out_ref[...] = _mm_math(emb_ref, wt_ref, b_ref)


def _tc_project_chunk(k, out_prev, emb_k, Wt, b2d):
    return pl.pallas_call(
        _mm_body,
        grid=(MM_STEPS,),
        in_specs=[
            pl.BlockSpec(memory_space=pl.ANY),
            pl.BlockSpec((MM_BLK2, DIM), lambda i: (i, 0)),
            pl.BlockSpec((DIM, DIM), lambda i: (0, 0)),
            pl.BlockSpec((1, DIM), lambda i: (0, 0)),
        ],
        out_specs=pl.BlockSpec(
            (2 * MM_BLK2, DIM), lambda i, k=k: (k * MM_STEPS + i, 0)
        ),
        out_shape=jax.ShapeDtypeStruct((N, DIM), jnp.float32),
        input_output_aliases={0: 0},
    )(out_prev, emb_k, Wt, b2d)


def _mm_first_body(emb_ref, wt_ref, b_ref, out_ref):
    out_ref[...] = _mm_math(emb_ref, wt_ref, b_ref)


def _tc_project_first(emb_k, Wt, b2d):
    return pl.pallas_call(
        _mm_first_body,
        grid=(MM_STEPS,),
        in_specs=[
            pl.BlockSpec((MM_BLK2, DIM), lambda i: (i, 0)),
            pl.BlockSpec((DIM, DIM), lambda i: (0, 0)),
            pl.BlockSpec((1, DIM), lambda i: (0, 0)),
        ],
        out_specs=pl.BlockSpec((2 * MM_BLK2, DIM), lambda i: (i, 0)),
        out_shape=jax.ShapeDtypeStruct((N, DIM), jnp.float32),
    )(emb_k, Wt, b2d)


def kernel(news_ids, news_categ, table, W, b):
    del news_categ
    ids = news_ids.T.reshape(K, NW, NCHUNK, CHUNK).astype(jnp.int32)
    Wt = W.T.astype(jnp.bfloat16)
    b2d = b.reshape(1, DIM)
    embs = [_gather_call(ids[k], table) for k in range(K)]
    out = _tc_project_first(embs[0], Wt, b2d)
    for k in range(1, K):
        out = _tc_project_chunk(k, out, embs[k], Wt, b2d)
    return out.reshape(L, B, DIM).transpose(1, 0, 2)


# speedup vs baseline: 1.0345x; 1.0345x over previous
import jax
import jax.numpy as jnp
from jax import lax
from jax.experimental import pallas as pl
from jax.experimental.pallas import tpu as pltpu
from jax.experimental.pallas import tpu_sc as plsc

NUM_EMB = 1000000
DIM = 128
B = 4096
L = 50
N = B * L

NC, NS = 2, 16
NW = NC * NS
K = 2
NK = N // K
ROWS_PER_W = NK // NW
CHUNK = 128
NCHUNK = ROWS_PER_W // CHUNK
NBUF = 5
LAG = NBUF - 1


def _sc_gather(ids_hbm, table_hbm, out_hbm, idx_v, rows_v, bf_v, g_sem, s_sem):
    wid = lax.axis_index("s") * NC + lax.axis_index("c")
    base2 = wid * (ROWS_PER_W // 2)
    pltpu.sync_copy(ids_hbm.at[wid], idx_v)
    g_h = [None] * NCHUNK
    s_h = [None] * NCHUNK

    def _convert(k):
        def body(rr, carry):
            for g in range(DIM // 16):
                a = rows_v[k, 2 * rr, pl.ds(16 * g, 16)]
                bb = rows_v[k, 2 * rr + 1, pl.ds(16 * g, 16)]
                y = plsc.pack(a, bb, format=plsc.PackFormat.INTERLEAVED)
                bf_v[k, rr, pl.ds(16 * g, 16)] = plsc.bitcast(y, jnp.float32)
            return carry

        lax.fori_loop(0, CHUNK // 2, body, None)

    for c in range(NCHUNK + LAG):
        if c < NCHUNK:
            g_h[c] = pltpu.async_copy(
                table_hbm.at[idx_v.at[c]], rows_v.at[c % NBUF], g_sem
            )
        d = c - LAG
        if 0 <= d < NCHUNK:
            if d >= NBUF:
                s_h[d - NBUF].wait()
            g_h[d].wait()
            _convert(d % NBUF)
            s_h[d] = pltpu.async_copy(
                bf_v.at[d % NBUF],
                out_hbm.at[pl.ds(base2 + d * (CHUNK // 2), CHUNK // 2)],
                s_sem,
            )
    for d in range(max(NCHUNK - NBUF, 0), NCHUNK):
        s_h[d].wait()


def _gather_call(ids, table):
    return pl.kernel(
        _sc_gather,
        mesh=plsc.VectorSubcoreMesh(
            core_axis_name="c", subcore_axis_name="s", num_cores=NC
        ),
        out_type=jax.ShapeDtypeStruct((NK // 2, DIM), jnp.float32),
        scratch_types=[
            pltpu.VMEM((NCHUNK, CHUNK), jnp.int32),
            pltpu.VMEM((NBUF, CHUNK, DIM), jnp.float32),
            pltpu.VMEM((NBUF, CHUNK // 2, DIM), jnp.float32),
            pltpu.SemaphoreType.DMA,
            pltpu.SemaphoreType.DMA,
        ],
        compiler_params=pltpu.CompilerParams(needs_layout_passes=False),
    )(ids, table)


MM_BLK2 = 2048
MM_STEPS = (NK // 2) // MM_BLK2


def _mm_math(emb_ref, wt_ref, b_ref):
    u = pltpu.bitcast(emb_ref[...], jnp.bfloat16)
    return (
        jnp.dot(u, wt_ref[...], preferred_element_type=jnp.float32) + b_ref[...]
    )


def _mm_body(prev_ref, emb_ref, wt_ref, b_ref, out_ref):
    del prev_ref
    out_ref[...] = _mm_math(emb_ref, wt_ref, b_ref)


def _tc_project_chunk(k, out_prev, emb_k, Wt, b2d):
    return pl.pallas_call(
        _mm_body,
        grid=(MM_STEPS,),
        in_specs=[
            pl.BlockSpec(memory_space=pl.ANY),
            pl.BlockSpec((MM_BLK2, DIM), lambda i: (i, 0)),
            pl.BlockSpec((DIM, DIM), lambda i: (0, 0)),
            pl.BlockSpec((1, DIM), lambda i: (0, 0)),
        ],
        out_specs=pl.BlockSpec(
            (2 * MM_BLK2, DIM), lambda i, k=k: (k * MM_STEPS + i, 0)
        ),
        out_shape=jax.ShapeDtypeStruct((N, DIM), jnp.float32),
        input_output_aliases={0: 0},
    )(out_prev, emb_k, Wt, b2d)


def _mm_first_body(emb_ref, wt_ref, b_ref, out_ref):
    out_ref[...] = _mm_math(emb_ref, wt_ref, b_ref)


def _tc_project_first(emb_k, Wt, b2d):
    return pl.pallas_call(
        _mm_first_body,
        grid=(MM_STEPS,),
        in_specs=[
            pl.BlockSpec((MM_BLK2, DIM), lambda i: (i, 0)),
            pl.BlockSpec((DIM, DIM), lambda i: (0, 0)),
            pl.BlockSpec((1, DIM), lambda i: (0, 0)),
        ],
        out_specs=pl.BlockSpec((2 * MM_BLK2, DIM), lambda i: (i, 0)),
        out_shape=jax.ShapeDtypeStruct((N, DIM), jnp.float32),
    )(emb_k, Wt, b2d)


def kernel(news_ids, news_categ, table, W, b):
    del news_categ
    ids = news_ids.T.reshape(K, NW, NCHUNK, CHUNK).astype(jnp.int32)
    Wt = W.T.astype(jnp.bfloat16)
    b2d = b.reshape(1, DIM)
    embs = [_gather_call(ids[k], table) for k in range(K)]
    out = _tc_project_first(embs[0], Wt, b2d)
    for k in range(1, K):
        out = _tc_project_chunk(k, out, embs[k], Wt, b2d)
    return out.reshape(L, B, DIM).transpose(1, 0, 2)


# speedup vs baseline: 1.1840x; 1.1445x over previous
import jax
import jax.numpy as jnp
from jax import lax
from jax.experimental import pallas as pl
from jax.experimental.pallas import tpu as pltpu
from jax.experimental.pallas import tpu_sc as plsc

NUM_EMB = 1000000
DIM = 128
B = 4096
L = 50
N = B * L

NC, NS = 2, 16
NW = NC * NS
CHUNK = 128
UNITS = (6, 12, 12, 12, 8)
K = len(UNITS)
NBUF = 7


def _make_sc_gather(nchunk):
    rows_per_w = nchunk * CHUNK

    def _sc_gather(ids_hbm, table_hbm, out_hbm, idx_v, rows_v, g_sem, s_sem):
        wid = lax.axis_index("s") * NC + lax.axis_index("c")
        base = wid * rows_per_w
        pltpu.sync_copy(ids_hbm.at[wid], idx_v)
        g_h = [None] * nchunk
        s_h = [None] * nchunk
        for c in range(nchunk + NBUF - 1):
            if c < nchunk:
                if c >= NBUF:
                    s_h[c - NBUF].wait()
                g_h[c] = pltpu.async_copy(
                    table_hbm.at[idx_v.at[c]], rows_v.at[c % NBUF], g_sem
                )
            d = c - (NBUF - 1)
            if 0 <= d < nchunk:
                g_h[d].wait()
                s_h[d] = pltpu.async_copy(
                    rows_v.at[d % NBUF],
                    out_hbm.at[pl.ds(base + d * CHUNK, CHUNK)],
                    s_sem,
                )
        for d in range(max(nchunk - NBUF, 0), nchunk):
            s_h[d].wait()

    return _sc_gather


def _gather_call(ids, table, nchunk):
    return pl.kernel(
        _make_sc_gather(nchunk),
        mesh=plsc.VectorSubcoreMesh(
            core_axis_name="c", subcore_axis_name="s", num_cores=NC
        ),
        out_type=jax.ShapeDtypeStruct((nchunk * CHUNK * NW, DIM), jnp.float32),
        scratch_types=[
            pltpu.VMEM((nchunk, CHUNK), jnp.int32),
            pltpu.VMEM((NBUF, CHUNK, DIM), jnp.float32),
            pltpu.SemaphoreType.DMA,
            pltpu.SemaphoreType.DMA,
        ],
    )(ids, table)


MM_BLK = 4096


def _mm_body(prev_ref, emb_ref, wt_ref, b_ref, out_ref):
    del prev_ref
    out_ref[...] = (
        jnp.dot(
            emb_ref[...].astype(jnp.bfloat16),
            wt_ref[...],
            preferred_element_type=jnp.float32,
        )
        + b_ref[...]
    )


def _tc_project_chunk(blk_off, steps, out_prev, emb_k, Wt, b2d):
    return pl.pallas_call(
        _mm_body,
        grid=(steps,),
        in_specs=[
            pl.BlockSpec(memory_space=pl.ANY),
            pl.BlockSpec((MM_BLK, DIM), lambda i: (i, 0)),
            pl.BlockSpec((DIM, DIM), lambda i: (0, 0)),
            pl.BlockSpec((1, DIM), lambda i: (0, 0)),
        ],
        out_specs=pl.BlockSpec((MM_BLK, DIM), lambda i, o=blk_off: (o + i, 0)),
        out_shape=jax.ShapeDtypeStruct((N, DIM), jnp.float32),
        input_output_aliases={0: 0},
    )(out_prev, emb_k, Wt, b2d)


def _mm_first_body(emb_ref, wt_ref, b_ref, out_ref):
    out_ref[...] = (
        jnp.dot(
            emb_ref[...].astype(jnp.bfloat16),
            wt_ref[...],
            preferred_element_type=jnp.float32,
        )
        + b_ref[...]
    )


def _tc_project_first(steps, emb_k, Wt, b2d):
    return pl.pallas_call(
        _mm_first_body,
        grid=(steps,),
        in_specs=[
            pl.BlockSpec((MM_BLK, DIM), lambda i: (i, 0)),
            pl.BlockSpec((DIM, DIM), lambda i: (0, 0)),
            pl.BlockSpec((1, DIM), lambda i: (0, 0)),
        ],
        out_specs=pl.BlockSpec((MM_BLK, DIM), lambda i: (i, 0)),
        out_shape=jax.ShapeDtypeStruct((N, DIM), jnp.float32),
    )(emb_k, Wt, b2d)


def kernel(news_ids, news_categ, table, W, b):
    del news_categ
    ids_flat = news_ids.T.reshape(-1).astype(jnp.int32)
    Wt = W.T.astype(jnp.bfloat16)
    b2d = b.reshape(1, DIM)
    embs = []
    row = 0
    for u in UNITS:
        nk = u * NW * CHUNK
        ids_k = lax.slice(ids_flat, (row,), (row + nk,)).reshape(NW, u, CHUNK)
        embs.append(_gather_call(ids_k, table, u))
        row += nk
    out = _tc_project_first(UNITS[0], embs[0], Wt, b2d)
    blk_off = UNITS[0]
    for k in range(1, K):
        out = _tc_project_chunk(blk_off, UNITS[k], out, embs[k], Wt, b2d)
        blk_off += UNITS[k]
    return out.reshape(L, B, DIM).transpose(1, 0, 2)


# speedup vs baseline: 1.1960x; 1.0101x over previous
import jax
import jax.numpy as jnp
from jax import lax
from jax.experimental import pallas as pl
from jax.experimental.pallas import tpu as pltpu
from jax.experimental.pallas import tpu_sc as plsc

NUM_EMB = 1000000
DIM = 128
B = 4096
L = 50
N = B * L

NC, NS = 2, 16
NW = NC * NS
K = 5
NK = N // K
ROWS_PER_W = NK // NW
CHUNK = 128
NCHUNK = ROWS_PER_W // CHUNK
NBUF = 6


def _sc_gather(ids_hbm, table_hbm, out_hbm, idx_v, rows_v, g_sem, s_sem):
    wid = lax.axis_index("s") * NC + lax.axis_index("c")
    base = wid * ROWS_PER_W
    pltpu.sync_copy(ids_hbm.at[wid], idx_v)
    g_h = [None] * NCHUNK
    s_h = [None] * NCHUNK
    for c in range(NCHUNK + NBUF - 1):
        if c < NCHUNK:
            if c >= NBUF:
                s_h[c - NBUF].wait()
            g_h[c] = pltpu.async_copy(
                table_hbm.at[idx_v.at[c]], rows_v.at[c % NBUF], g_sem
            )
        d = c - (NBUF - 1)
        if 0 <= d < NCHUNK:
            g_h[d].wait()
            s_h[d] = pltpu.async_copy(
                rows_v.at[d % NBUF],
                out_hbm.at[pl.ds(base + d * CHUNK, CHUNK)],
                s_sem,
            )
    for d in range(max(NCHUNK - NBUF, 0), NCHUNK):
        s_h[d].wait()


def _gather_call(ids, table):
    return pl.kernel(
        _sc_gather,
        mesh=plsc.VectorSubcoreMesh(
            core_axis_name="c", subcore_axis_name="s", num_cores=NC
        ),
        out_type=jax.ShapeDtypeStruct((NK, DIM), jnp.float32),
        scratch_types=[
            pltpu.VMEM((NCHUNK, CHUNK), jnp.int32),
            pltpu.VMEM((NBUF, CHUNK, DIM), jnp.float32),
            pltpu.SemaphoreType.DMA,
            pltpu.SemaphoreType.DMA,
        ],
    )(ids, table)


MM_BLK = 4096
MM_STEPS = NK // MM_BLK


def _mm_body(prev_ref, emb_ref, wt_ref, b_ref, out_ref):
    del prev_ref
    out_ref[...] = (
        jnp.dot(
            emb_ref[...].astype(jnp.bfloat16),
            wt_ref[...],
            preferred_element_type=jnp.float32,
        )
        + b_ref[...]
    )


def _tc_project_chunk(k, out_prev, emb_k, Wt, b2d):
    return pl.pallas_call(
        _mm_body,
        grid=(MM_STEPS,),
        in_specs=[
            pl.BlockSpec(memory_space=pl.ANY),
            pl.BlockSpec((MM_BLK, DIM), lambda i: (i, 0)),
            pl.BlockSpec((DIM, DIM), lambda i: (0, 0)),
            pl.BlockSpec((1, DIM), lambda i: (0, 0)),
        ],
        out_specs=pl.BlockSpec((MM_BLK, DIM), lambda i, k=k: (k * MM_STEPS + i, 0)),
        out_shape=jax.ShapeDtypeStruct((N, DIM), jnp.float32),
        input_output_aliases={0: 0},
    )(out_prev, emb_k, Wt, b2d)


def _mm_first_body(emb_ref, wt_ref, b_ref, out_ref):
    out_ref[...] = (
        jnp.dot(
            emb_ref[...].astype(jnp.bfloat16),
            wt_ref[...],
            preferred_element_type=jnp.float32,
        )
        + b_ref[...]
    )


def _tc_project_first(emb_k, Wt, b2d):
    return pl.pallas_call(
        _mm_first_body,
        grid=(MM_STEPS,),
        in_specs=[
            pl.BlockSpec((MM_BLK, DIM), lambda i: (i, 0)),
            pl.BlockSpec((DIM, DIM), lambda i: (0, 0)),
            pl.BlockSpec((1, DIM), lambda i: (0, 0)),
        ],
        out_specs=pl.BlockSpec((MM_BLK, DIM), lambda i: (i, 0)),
        out_shape=jax.ShapeDtypeStruct((N, DIM), jnp.float32),
    )(emb_k, Wt, b2d)


def kernel(news_ids, news_categ, table, W, b):
    del news_categ
    ids = news_ids.T.reshape(K, NW, NCHUNK, CHUNK).astype(jnp.int32)
    Wt = W.T.astype(jnp.bfloat16)
    b2d = b.reshape(1, DIM)
    embs = [_gather_call(ids[k], table) for k in range(K)]
    out = _tc_project_first(embs[0], Wt, b2d)
    for k in range(1, K):
        out = _tc_project_chunk(k, out, embs[k], Wt, b2d)
    return out.reshape(L, B, DIM).transpose(1, 0, 2)
